# baseline (speedup 1.0000x reference)
import functools

import jax
import jax.numpy as jnp
from jax import lax
from jax.experimental import pallas as pl
from jax.experimental.pallas import tpu as pltpu
from jax.experimental.pallas import tpu_sc as plsc

VOCAB = 1_000_000
D = 64
L = 20
NC = 2
NS = 16
NW = NC * NS
C = 128
G = C // 16


def _cbow_body(b_per_w, n_chunks,
               ctxT_hbm, cen_hbm, ctab_hbm, otab_hbm, out_hbm,
               idxT, idxD, cidx, cidxD, buf, accT, cbuf, r0v, score_v,
               sem_a, sem_b, sem_c):
    cid = lax.axis_index("c")
    sid = lax.axis_index("s")
    wid = sid * NC + cid
    base = wid * b_per_w
    iota16 = lax.iota(jnp.int32, 16)

    pltpu.sync_copy(ctab_hbm.at[0, pl.ds(0, D)], r0v)

    def chunk_body(k, _):
        cb = base + k * C

        pltpu.sync_copy(ctxT_hbm.at[:, pl.ds(cb, C)], idxT)
        pltpu.sync_copy(cen_hbm.at[pl.ds(cb, C)], cidx)

        def idx_body(g, _):
            sl = pl.ds(g * 16, 16)
            for j in range(L):
                idxD[j, sl] = lax.shift_right_logical(idxT[j, sl], 1)
            cidxD[sl] = lax.shift_right_logical(cidx[sl], 1)
            return 0
        lax.fori_loop(0, G, idx_body, 0)

        ce_cp = pltpu.async_copy(otab_hbm.at[cidxD], cbuf, sem_c)

        cps = [None, None]
        cps[0] = pltpu.async_copy(ctab_hbm.at[idxD.at[0]], buf.at[0], sem_a)
        for j in range(L):
            p = j % 2
            if j + 1 < L:
                pn = (j + 1) % 2
                cps[pn] = pltpu.async_copy(
                    ctab_hbm.at[idxD.at[j + 1]], buf.at[pn],
                    sem_b if pn else sem_a)
            cps[p].wait()

            bufp = buf.at[p]
            halfs = []
            for g in range(G):
                sl = pl.ds(g * 16, 16)
                halfs.append((idxT[j, sl] & 1) * 64)

            def d_body(d, _, j=j, bufp=bufp, halfs=halfs):
                dsp = jnp.full((16,), 0, jnp.int32) + d
                for g in range(G):
                    sl = pl.ds(g * 16, 16)
                    colB = plsc.load_gather(
                        bufp, [g * 16 + iota16, halfs[g] + dsp])
                    if j == 0:
                        accT[d, sl] = colB
                    else:
                        plsc.addupdate(accT.at[d, sl], colB)
                return 0
            lax.fori_loop(0, D, d_body, 0)

        ce_cp.wait()

        def grp_body(g, _):
            sl = pl.ds(g * 16, 16)
            rows = g * 16 + iota16
            chalf = (cidx[sl] & 1) * 64

            nz = jnp.zeros((16,), jnp.float32)
            for j in range(L):
                nz = nz + jnp.where(idxT[j, sl] == 0, 1.0, 0.0).astype(
                    jnp.float32)

            def dd_body(d, carry):
                A, Bv = carry
                dsp = jnp.full((16,), 0, jnp.int32) + d
                a16 = accT[d, sl]
                colC = plsc.load_gather(cbuf, [rows, chalf + dsp])
                r0d = plsc.load_gather(r0v, [dsp])
                return A + a16 * colC, Bv + r0d * colC

            zero = jnp.zeros((16,), jnp.float32)
            A, Bv = lax.fori_loop(0, D, dd_body, (zero, zero))
            cnt = jnp.float32(L) - nz
            sc = (A - nz * Bv) / (cnt + 1e-8)
            sc = jnp.where(nz >= jnp.float32(L), 0.0, sc)
            score_v[sl] = sc
            return 0
        lax.fori_loop(0, G, grp_body, 0)

        pltpu.sync_copy(score_v, out_hbm.at[pl.ds(cb, C)])
        return 0

    lax.fori_loop(0, n_chunks, chunk_body, 0)


@jax.jit
def _cbow_sc(ctxT, center, ctab2, otab2):
    B = ctxT.shape[1]
    b_per_w = B // NW
    n_chunks = b_per_w // C
    mesh = plsc.VectorSubcoreMesh(core_axis_name="c", subcore_axis_name="s")

    kern = pl.kernel(
        functools.partial(_cbow_body, b_per_w, n_chunks),
        out_type=jax.ShapeDtypeStruct((B,), jnp.float32),
        mesh=mesh,
        compiler_params=pltpu.CompilerParams(needs_layout_passes=False),
        scratch_types=[
            pltpu.VMEM((L, C), jnp.int32),
            pltpu.VMEM((L, C), jnp.int32),
            pltpu.VMEM((C,), jnp.int32),
            pltpu.VMEM((C,), jnp.int32),
            pltpu.VMEM((2, C, 2 * D), jnp.float32),
            pltpu.VMEM((D, C), jnp.float32),
            pltpu.VMEM((C, 2 * D), jnp.float32),
            pltpu.VMEM((D,), jnp.float32),
            pltpu.VMEM((C,), jnp.float32),
            pltpu.SemaphoreType.DMA,
            pltpu.SemaphoreType.DMA,
            pltpu.SemaphoreType.DMA,
        ],
    )
    return kern(ctxT, center, ctab2, otab2)


def kernel(context_words, center, context_table, output_table):
    ctxT = context_words.astype(jnp.int32).T
    ctab2 = context_table.reshape(VOCAB // 2, 2 * D)
    otab2 = output_table.reshape(VOCAB // 2, 2 * D)
    return _cbow_sc(ctxT, center.astype(jnp.int32), ctab2, otab2)


# speedup vs baseline: 1.2766x; 1.2766x over previous
import functools

import jax
import jax.numpy as jnp
from jax import lax
from jax.experimental import pallas as pl
from jax.experimental.layout import Format, Layout
from jax.experimental.pallas import tpu as pltpu
from jax.experimental.pallas import tpu_sc as plsc

VOCAB = 1_000_000
D = 64
L = 20
NC = 2
NS = 16
NW = NC * NS
C = 128
G = C // 16
AW = 65


def _cbow_body(b_per_w, n_chunks,
               ctxT_hbm, cen_hbm, ctab_hbm, otab_hbm, out_hbm,
               idxT, cidx, buf, acc, cbuf, cb65, r0v, score_v,
               sem_a, sem_b, sem_c):
    cid = lax.axis_index("c")
    sid = lax.axis_index("s")
    wid = sid * NC + cid
    base = wid * b_per_w
    iota16 = lax.iota(jnp.int32, 16)

    pltpu.sync_copy(ctab_hbm.at[0], r0v)

    def chunk_body(k, _):
        cb = base + k * C

        pltpu.sync_copy(ctxT_hbm.at[:, pl.ds(cb, C)], idxT)
        pltpu.sync_copy(cen_hbm.at[pl.ds(cb, C)], cidx)

        ce_cp = pltpu.async_copy(otab_hbm.at[cidx], cbuf, sem_c)

        cps = [None, None]
        cps[0] = pltpu.async_copy(ctab_hbm.at[idxT.at[0]], buf.at[0], sem_a)
        for j in range(L):
            p = j % 2
            if j + 1 < L:
                pn = (j + 1) % 2
                cps[pn] = pltpu.async_copy(
                    ctab_hbm.at[idxT.at[j + 1]], buf.at[pn],
                    sem_b if pn else sem_a)
            cps[p].wait()

            def acc_body(r, _, j=j, p=p):
                for c4 in range(D // 16):
                    sl = pl.ds(c4 * 16, 16)
                    if j == 0:
                        acc[r, sl] = buf[p, r, sl]
                    else:
                        plsc.addupdate(acc.at[r, sl], buf[p, r, sl])
                return 0
            lax.fori_loop(0, C, acc_body, 0)

        ce_cp.wait()

        def stage_body(r, _):
            for c4 in range(D // 16):
                sl = pl.ds(c4 * 16, 16)
                cb65[r, sl] = cbuf[r, sl]
            return 0
        lax.fori_loop(0, C, stage_body, 0)

        def grp_body(g, _):
            sl = pl.ds(g * 16, 16)
            rows = g * 16 + iota16

            nz = jnp.zeros((16,), jnp.float32)
            for j in range(L):
                nz = nz + jnp.where(idxT[j, sl] == 0, 1.0, 0.0).astype(
                    jnp.float32)

            def dd_body(d, carry):
                A, Bv = carry
                dsp = jnp.full((16,), 0, jnp.int32) + d
                a16 = plsc.load_gather(acc, [rows, dsp])
                c16 = plsc.load_gather(cb65, [rows, dsp])
                r0d = plsc.load_gather(r0v, [dsp])
                return A + a16 * c16, Bv + r0d * c16

            zero = jnp.zeros((16,), jnp.float32)
            A, Bv = lax.fori_loop(0, D, dd_body, (zero, zero))
            cnt = jnp.float32(L) - nz
            sc = (A - nz * Bv) / (cnt + 1e-8)
            sc = jnp.where(nz >= jnp.float32(L), 0.0, sc)
            score_v[sl] = sc
            return 0
        lax.fori_loop(0, G, grp_body, 0)

        pltpu.sync_copy(score_v, out_hbm.at[pl.ds(cb, C)])
        return 0

    lax.fori_loop(0, n_chunks, chunk_body, 0)


@jax.jit
def _cbow_sc(ctxT, center, context_table, output_table):
    B = ctxT.shape[1]
    b_per_w = B // NW
    n_chunks = b_per_w // C
    mesh = plsc.VectorSubcoreMesh(core_axis_name="c", subcore_axis_name="s")

    kern = pl.kernel(
        functools.partial(_cbow_body, b_per_w, n_chunks),
        out_type=jax.ShapeDtypeStruct((B,), jnp.float32),
        mesh=mesh,
        compiler_params=pltpu.CompilerParams(
            needs_layout_passes=False, use_tc_tiling_on_sc=False),
        scratch_types=[
            pltpu.VMEM((L, C), jnp.int32),
            pltpu.VMEM((C,), jnp.int32),
            pltpu.VMEM((2, C, D), jnp.float32),
            pltpu.VMEM((C, AW), jnp.float32),
            pltpu.VMEM((C, D), jnp.float32),
            pltpu.VMEM((C, AW), jnp.float32),
            pltpu.VMEM((D,), jnp.float32),
            pltpu.VMEM((C,), jnp.float32),
            pltpu.SemaphoreType.DMA,
            pltpu.SemaphoreType.DMA,
            pltpu.SemaphoreType.DMA,
        ],
    )
    return kern(ctxT, center, context_table, output_table)


_SC_LAYOUT = Layout(major_to_minor=(0, 1), tiling=((16,),))


def kernel(context_words, center, context_table, output_table):
    ctxT = context_words.astype(jnp.int32).T
    fmt = Format(_SC_LAYOUT,
                 jax.sharding.SingleDeviceSharding(jax.devices()[0]))
    ctab_sc = jax.device_put(context_table, fmt)
    otab_sc = jax.device_put(output_table, fmt)
    return _cbow_sc(ctxT, center.astype(jnp.int32), ctab_sc, otab_sc)


# speedup vs baseline: 1.4268x; 1.1177x over previous
import functools

import jax
import jax.numpy as jnp
from jax import lax
from jax.experimental import pallas as pl
from jax.experimental.pallas import tpu as pltpu
from jax.experimental.pallas import tpu_sc as plsc

VOCAB = 1_000_000
D = 64
L = 20
NC = 2
NS = 16
NW = NC * NS
C = 128

_PARAMS = pltpu.CompilerParams(
    needs_layout_passes=False, use_tc_tiling_on_sc=False)
_MESH = dict(core_axis_name="c", subcore_axis_name="s")


def _ctx_body(b_per_w, n_chunks,
              ctxT_hbm, ctab_hbm, acc_hbm, nz_hbm, r0_hbm,
              idxT, buf, acc, nzv, r0v, sem_a, sem_b):
    cid = lax.axis_index("c")
    sid = lax.axis_index("s")
    wid = sid * NC + cid
    base = wid * b_per_w

    pltpu.sync_copy(ctab_hbm.at[0], r0v)

    @pl.when(wid == 0)
    def _():
        pltpu.sync_copy(r0v, r0_hbm)

    for k in range(n_chunks):
        cb = base + k * C

        pltpu.sync_copy(ctxT_hbm.at[:, pl.ds(cb, C)], idxT)

        def zero_body(r, _):
            for c4 in range(D // 16):
                acc[r, pl.ds(c4 * 16, 16)] = jnp.zeros((16,), jnp.float32)
            return 0
        lax.fori_loop(0, C, zero_body, 0)

        cps = [None, None]
        cps[0] = pltpu.async_copy(ctab_hbm.at[idxT.at[0]], buf.at[0], sem_a)
        for j in range(L):
            p = j % 2
            if j + 1 < L:
                pn = (j + 1) % 2
                cps[pn] = pltpu.async_copy(
                    ctab_hbm.at[idxT.at[j + 1]], buf.at[pn],
                    sem_b if pn else sem_a)
            cps[p].wait()

            def acc_body(r, _, p=p):
                for c4 in range(D // 16):
                    sl = pl.ds(c4 * 16, 16)
                    plsc.addupdate(acc.at[r, sl], buf[p, r, sl])
                return 0
            lax.fori_loop(0, C, acc_body, 0)

        def nz_body(g, _):
            sl = pl.ds(g * 16, 16)
            nz = jnp.zeros((16,), jnp.float32)
            for j in range(L):
                colj = idxT[j, sl]
                nz = nz + jnp.where(colj == 0, 1.0, 0.0).astype(jnp.float32)
            nzv[sl] = nz
            return 0
        lax.fori_loop(0, C // 16, nz_body, 0)

        pltpu.sync_copy(acc, acc_hbm.at[pl.ds(cb, C), :])
        pltpu.sync_copy(nzv, nz_hbm.at[pl.ds(cb, C)])


def _dot_body(b_per_w, n_chunks,
              cen_hbm, otab_hbm, acc_hbm, nz_hbm, r0_hbm, out_hbm,
              cidx, acc, cbuf, nzv, r0v, score_v, sem_c):
    cid = lax.axis_index("c")
    sid = lax.axis_index("s")
    wid = sid * NC + cid
    base = wid * b_per_w

    pltpu.sync_copy(r0_hbm, r0v)

    for k in range(n_chunks):
        cb = base + k * C

        pltpu.sync_copy(cen_hbm.at[pl.ds(cb, C)], cidx)
        ce_cp = pltpu.async_copy(otab_hbm.at[cidx], cbuf, sem_c)
        pltpu.sync_copy(acc_hbm.at[pl.ds(cb, C), :], acc)
        pltpu.sync_copy(nz_hbm.at[pl.ds(cb, C)], nzv)
        ce_cp.wait()

        def grp_body(g, _):
            sl = pl.ds(g * 16, 16)
            rows = g * 16 + lax.iota(jnp.int32, 16)
            nz = nzv[sl]

            def d_body(d, carry):
                A, Bv = carry
                dsp = jnp.full((16,), d, jnp.int32)
                colA = plsc.load_gather(acc, [rows, dsp])
                colC = plsc.load_gather(cbuf, [rows, dsp])
                r0d = plsc.load_gather(r0v, [dsp])
                return A + colA * colC, Bv + r0d * colC

            zero = jnp.zeros((16,), jnp.float32)
            A, Bv = lax.fori_loop(0, D, d_body, (zero, zero))
            cnt = jnp.float32(L) - nz
            sc = (A - nz * Bv) / (cnt + 1e-8)
            sc = jnp.where(nz >= jnp.float32(L), 0.0, sc)
            score_v[sl] = sc
            return 0
        lax.fori_loop(0, C // 16, grp_body, 0)

        pltpu.sync_copy(score_v, out_hbm.at[pl.ds(cb, C)])


@jax.jit
def _cbow_sc(ctxT, center, context_table, output_table):
    B = ctxT.shape[1]
    b_per_w = B // NW
    n_chunks = b_per_w // C

    ctx_kern = pl.kernel(
        functools.partial(_ctx_body, b_per_w, n_chunks),
        out_type=(
            jax.ShapeDtypeStruct((B, D), jnp.float32),
            jax.ShapeDtypeStruct((B,), jnp.float32),
            jax.ShapeDtypeStruct((D,), jnp.float32),
        ),
        mesh=plsc.VectorSubcoreMesh(**_MESH),
        compiler_params=_PARAMS,
        scratch_types=[
            pltpu.VMEM((L, C), jnp.int32),
            pltpu.VMEM((2, C, D), jnp.float32),
            pltpu.VMEM((C, D), jnp.float32),
            pltpu.VMEM((C,), jnp.float32),
            pltpu.VMEM((D,), jnp.float32),
            pltpu.SemaphoreType.DMA,
            pltpu.SemaphoreType.DMA,
        ],
    )
    acc_p, nz_p, r0_p = ctx_kern(ctxT, context_table)

    dot_kern = pl.kernel(
        functools.partial(_dot_body, b_per_w, n_chunks),
        out_type=jax.ShapeDtypeStruct((B,), jnp.float32),
        mesh=plsc.VectorSubcoreMesh(**_MESH),
        compiler_params=_PARAMS,
        scratch_types=[
            pltpu.VMEM((C,), jnp.int32),
            pltpu.VMEM((C, D), jnp.float32),
            pltpu.VMEM((C, D), jnp.float32),
            pltpu.VMEM((C,), jnp.float32),
            pltpu.VMEM((D,), jnp.float32),
            pltpu.VMEM((C,), jnp.float32),
            pltpu.SemaphoreType.DMA,
        ],
    )
    return dot_kern(center, output_table, acc_p, nz_p, r0_p)


def kernel(context_words, center, context_table, output_table):
    ctxT = context_words.astype(jnp.int32).T
    return _cbow_sc(ctxT, center.astype(jnp.int32),
                    context_table, output_table)


# speedup vs baseline: 1.4361x; 1.0065x over previous
import functools

import jax
import jax.numpy as jnp
from jax import lax
from jax.experimental import pallas as pl
from jax.experimental.pallas import tpu as pltpu
from jax.experimental.pallas import tpu_sc as plsc

VOCAB = 1_000_000
D = 64
L = 20
NC = 2
NS = 16
NW = NC * NS
C = 128

_PARAMS = pltpu.CompilerParams(
    needs_layout_passes=False, use_tc_tiling_on_sc=False)
_MESH = dict(core_axis_name="c", subcore_axis_name="s")


def _ctx_body(b_per_w, n_chunks,
              ctxT_hbm, ctab_hbm, acc_hbm, nz_hbm, r0_hbm,
              idxT, buf, acc, nzv, r0v, sem_a, sem_b):
    cid = lax.axis_index("c")
    sid = lax.axis_index("s")
    wid = sid * NC + cid
    base = wid * b_per_w

    pltpu.sync_copy(ctab_hbm.at[0], r0v)

    @pl.when(wid == 0)
    def _():
        pltpu.sync_copy(r0v, r0_hbm)

    for k in range(n_chunks):
        cb = base + k * C

        pltpu.sync_copy(ctxT_hbm.at[:, pl.ds(cb, C)], idxT)

        def zero_body(r, _):
            for c4 in range(D // 16):
                acc[r, pl.ds(c4 * 16, 16)] = jnp.zeros((16,), jnp.float32)
            return 0
        lax.fori_loop(0, C, zero_body, 0)

        cps = [None, None]
        cps[0] = pltpu.async_copy(ctab_hbm.at[idxT.at[0]], buf.at[0], sem_a)
        for j in range(L):
            p = j % 2
            if j + 1 < L:
                pn = (j + 1) % 2
                cps[pn] = pltpu.async_copy(
                    ctab_hbm.at[idxT.at[j + 1]], buf.at[pn],
                    sem_b if pn else sem_a)
            cps[p].wait()

            def acc_body(r, _, p=p):
                for c4 in range(D // 16):
                    sl = pl.ds(c4 * 16, 16)
                    plsc.addupdate(acc.at[r, sl], buf[p, r, sl])
                return 0
            lax.fori_loop(0, C, acc_body, 0)

        def nz_body(g, _):
            sl = pl.ds(g * 16, 16)
            nz = jnp.zeros((16,), jnp.float32)
            for j in range(L):
                colj = idxT[j, sl]
                nz = nz + jnp.where(colj == 0, 1.0, 0.0).astype(jnp.float32)
            nzv[sl] = nz
            return 0
        lax.fori_loop(0, C // 16, nz_body, 0)

        pltpu.sync_copy(acc, acc_hbm.at[pl.ds(cb, C), :])
        pltpu.sync_copy(nzv, nz_hbm.at[pl.ds(cb, C)])


def _dot_body(b_per_w, n_chunks,
              cen_hbm, otab_hbm, acc_hbm, nz_hbm, r0_hbm, out_hbm,
              cidx, acc, cbuf, a65, c65, nzv, r0v, score_v, sem_c):
    cid = lax.axis_index("c")
    sid = lax.axis_index("s")
    wid = sid * NC + cid
    base = wid * b_per_w

    pltpu.sync_copy(r0_hbm, r0v)

    for k in range(n_chunks):
        cb = base + k * C

        pltpu.sync_copy(cen_hbm.at[pl.ds(cb, C)], cidx)
        ce_cp = pltpu.async_copy(otab_hbm.at[cidx], cbuf, sem_c)
        pltpu.sync_copy(acc_hbm.at[pl.ds(cb, C), :], acc)
        pltpu.sync_copy(nz_hbm.at[pl.ds(cb, C)], nzv)
        ce_cp.wait()

        def stage_body(r, _):
            for c4 in range(D // 16):
                sl = pl.ds(c4 * 16, 16)
                a65[r, sl] = acc[r, sl]
                c65[r, sl] = cbuf[r, sl]
            return 0
        lax.fori_loop(0, C, stage_body, 0)

        def grp_body(g, _):
            sl = pl.ds(g * 16, 16)
            rows = g * 16 + lax.iota(jnp.int32, 16)
            nz = nzv[sl]

            def d_body(d, carry):
                A, Bv = carry
                dsp = jnp.full((16,), d, jnp.int32)
                colA = plsc.load_gather(a65, [rows, dsp])
                colC = plsc.load_gather(c65, [rows, dsp])
                r0d = plsc.load_gather(r0v, [dsp])
                return A + colA * colC, Bv + r0d * colC

            zero = jnp.zeros((16,), jnp.float32)
            A, Bv = lax.fori_loop(0, D, d_body, (zero, zero))
            cnt = jnp.float32(L) - nz
            sc = (A - nz * Bv) / (cnt + 1e-8)
            sc = jnp.where(nz >= jnp.float32(L), 0.0, sc)
            score_v[sl] = sc
            return 0
        lax.fori_loop(0, C // 16, grp_body, 0)

        pltpu.sync_copy(score_v, out_hbm.at[pl.ds(cb, C)])


@jax.jit
def _cbow_sc(ctxT, center, context_table, output_table):
    B = ctxT.shape[1]
    b_per_w = B // NW
    n_chunks = b_per_w // C

    ctx_kern = pl.kernel(
        functools.partial(_ctx_body, b_per_w, n_chunks),
        out_type=(
            jax.ShapeDtypeStruct((B, D), jnp.float32),
            jax.ShapeDtypeStruct((B,), jnp.float32),
            jax.ShapeDtypeStruct((D,), jnp.float32),
        ),
        mesh=plsc.VectorSubcoreMesh(**_MESH),
        compiler_params=_PARAMS,
        scratch_types=[
            pltpu.VMEM((L, C), jnp.int32),
            pltpu.VMEM((2, C, D), jnp.float32),
            pltpu.VMEM((C, D), jnp.float32),
            pltpu.VMEM((C,), jnp.float32),
            pltpu.VMEM((D,), jnp.float32),
            pltpu.SemaphoreType.DMA,
            pltpu.SemaphoreType.DMA,
        ],
    )
    acc_p, nz_p, r0_p = ctx_kern(ctxT, context_table)

    dot_kern = pl.kernel(
        functools.partial(_dot_body, b_per_w, n_chunks),
        out_type=jax.ShapeDtypeStruct((B,), jnp.float32),
        mesh=plsc.VectorSubcoreMesh(**_MESH),
        compiler_params=_PARAMS,
        scratch_types=[
            pltpu.VMEM((C,), jnp.int32),
            pltpu.VMEM((C, D), jnp.float32),
            pltpu.VMEM((C, D), jnp.float32),
            pltpu.VMEM((C, D + 1), jnp.float32),
            pltpu.VMEM((C, D + 1), jnp.float32),
            pltpu.VMEM((C,), jnp.float32),
            pltpu.VMEM((D,), jnp.float32),
            pltpu.VMEM((C,), jnp.float32),
            pltpu.SemaphoreType.DMA,
        ],
    )
    return dot_kern(center, output_table, acc_p, nz_p, r0_p)


def kernel(context_words, center, context_table, output_table):
    ctxT = context_words.astype(jnp.int32).T
    return _cbow_sc(ctxT, center.astype(jnp.int32),
                    context_table, output_table)


# speedup vs baseline: 1.4395x; 1.0024x over previous
import functools

import jax
import jax.numpy as jnp
from jax import lax
from jax.experimental import pallas as pl
from jax.experimental.pallas import tpu as pltpu
from jax.experimental.pallas import tpu_sc as plsc

VOCAB = 1_000_000
D = 64
L = 20
NC = 2
NS = 16
NW = NC * NS
C = 128

_PARAMS = pltpu.CompilerParams(
    needs_layout_passes=False, use_tc_tiling_on_sc=False)
_MESH = dict(core_axis_name="c", subcore_axis_name="s")


def _ctx_body(b_per_w, n_chunks,
              ctxT_hbm, ctab_hbm, acc_hbm, nz_hbm, r0_hbm,
              idxT, buf, acc, nzv, r0v, sem_a, sem_b):
    cid = lax.axis_index("c")
    sid = lax.axis_index("s")
    wid = sid * NC + cid
    base = wid * b_per_w

    pltpu.sync_copy(ctab_hbm.at[0], r0v)

    @pl.when(wid == 0)
    def _():
        pltpu.sync_copy(r0v, r0_hbm)

    for k in range(n_chunks):
        cb = base + k * C

        pltpu.sync_copy(ctxT_hbm.at[:, pl.ds(cb, C)], idxT)

        def zero_body(r, _):
            for c4 in range(D // 16):
                acc[r, pl.ds(c4 * 16, 16)] = jnp.zeros((16,), jnp.float32)
            return 0
        lax.fori_loop(0, C, zero_body, 0)

        cps = [None, None]
        cps[0] = pltpu.async_copy(ctab_hbm.at[idxT.at[0]], buf.at[0], sem_a)
        for j in range(L):
            p = j % 2
            if j + 1 < L:
                pn = (j + 1) % 2
                cps[pn] = pltpu.async_copy(
                    ctab_hbm.at[idxT.at[j + 1]], buf.at[pn],
                    sem_b if pn else sem_a)
            cps[p].wait()

            def acc_body(r, _, p=p):
                for c4 in range(D // 16):
                    sl = pl.ds(c4 * 16, 16)
                    plsc.addupdate(acc.at[r, sl], buf[p, r, sl])
                return 0
            lax.fori_loop(0, C, acc_body, 0)

        def nz_body(g, _):
            sl = pl.ds(g * 16, 16)
            nz = jnp.zeros((16,), jnp.float32)
            for j in range(L):
                colj = idxT[j, sl]
                nz = nz + jnp.where(colj == 0, 1.0, 0.0).astype(jnp.float32)
            nzv[sl] = nz
            return 0
        lax.fori_loop(0, C // 16, nz_body, 0)

        pltpu.sync_copy(acc, acc_hbm.at[pl.ds(cb, C), :])
        pltpu.sync_copy(nzv, nz_hbm.at[pl.ds(cb, C)])


def _dot_body(b_per_w, n_chunks,
              cen_hbm, otab_hbm, acc_hbm, nz_hbm, r0_hbm, out_hbm,
              cidx, acc, cbuf, a65, c65, nzv, r0v, score_v, sem_c):
    cid = lax.axis_index("c")
    sid = lax.axis_index("s")
    wid = sid * NC + cid
    base = wid * b_per_w

    pltpu.sync_copy(r0_hbm, r0v)

    for k in range(n_chunks):
        cb = base + k * C

        pltpu.sync_copy(cen_hbm.at[pl.ds(cb, C)], cidx)
        ce_cp = pltpu.async_copy(otab_hbm.at[cidx], cbuf, sem_c)
        pltpu.sync_copy(acc_hbm.at[pl.ds(cb, C), :], acc)
        pltpu.sync_copy(nz_hbm.at[pl.ds(cb, C)], nzv)
        ce_cp.wait()

        def stage_body(r, _):
            for c4 in range(D // 16):
                sl = pl.ds(c4 * 16, 16)
                a65[r, sl] = acc[r, sl]
                c65[r, sl] = cbuf[r, sl]
            return 0
        lax.fori_loop(0, C, stage_body, 0, unroll=4)

        def grp_body(g, _):
            sl = pl.ds(g * 16, 16)
            rows = g * 16 + lax.iota(jnp.int32, 16)
            nz = nzv[sl]

            def d_body(d, carry):
                A, Bv = carry
                dsp = jnp.full((16,), d, jnp.int32)
                colA = plsc.load_gather(a65, [rows, dsp])
                colC = plsc.load_gather(c65, [rows, dsp])
                r0d = plsc.load_gather(r0v, [dsp])
                return A + colA * colC, Bv + r0d * colC

            zero = jnp.zeros((16,), jnp.float32)
            A, Bv = lax.fori_loop(0, D, d_body, (zero, zero), unroll=8)
            cnt = jnp.float32(L) - nz
            sc = (A - nz * Bv) / (cnt + 1e-8)
            sc = jnp.where(nz >= jnp.float32(L), 0.0, sc)
            score_v[sl] = sc
            return 0
        lax.fori_loop(0, C // 16, grp_body, 0)

        pltpu.sync_copy(score_v, out_hbm.at[pl.ds(cb, C)])


@jax.jit
def _cbow_sc(ctxT, center, context_table, output_table):
    B = ctxT.shape[1]
    b_per_w = B // NW
    n_chunks = b_per_w // C

    ctx_kern = pl.kernel(
        functools.partial(_ctx_body, b_per_w, n_chunks),
        out_type=(
            jax.ShapeDtypeStruct((B, D), jnp.float32),
            jax.ShapeDtypeStruct((B,), jnp.float32),
            jax.ShapeDtypeStruct((D,), jnp.float32),
        ),
        mesh=plsc.VectorSubcoreMesh(**_MESH),
        compiler_params=_PARAMS,
        scratch_types=[
            pltpu.VMEM((L, C), jnp.int32),
            pltpu.VMEM((2, C, D), jnp.float32),
            pltpu.VMEM((C, D), jnp.float32),
            pltpu.VMEM((C,), jnp.float32),
            pltpu.VMEM((D,), jnp.float32),
            pltpu.SemaphoreType.DMA,
            pltpu.SemaphoreType.DMA,
        ],
    )
    acc_p, nz_p, r0_p = ctx_kern(ctxT, context_table)

    dot_kern = pl.kernel(
        functools.partial(_dot_body, b_per_w, n_chunks),
        out_type=jax.ShapeDtypeStruct((B,), jnp.float32),
        mesh=plsc.VectorSubcoreMesh(**_MESH),
        compiler_params=_PARAMS,
        scratch_types=[
            pltpu.VMEM((C,), jnp.int32),
            pltpu.VMEM((C, D), jnp.float32),
            pltpu.VMEM((C, D), jnp.float32),
            pltpu.VMEM((C, D + 1), jnp.float32),
            pltpu.VMEM((C, D + 1), jnp.float32),
            pltpu.VMEM((C,), jnp.float32),
            pltpu.VMEM((D,), jnp.float32),
            pltpu.VMEM((C,), jnp.float32),
            pltpu.SemaphoreType.DMA,
        ],
    )
    return dot_kern(center, output_table, acc_p, nz_p, r0_p)


def kernel(context_words, center, context_table, output_table):
    ctxT = context_words.astype(jnp.int32).T
    return _cbow_sc(ctxT, center.astype(jnp.int32),
                    context_table, output_table)
